# baseline (speedup 1.0000x reference)
import functools

import jax
import jax.numpy as jnp
from jax import lax
from jax.experimental import pallas as pl
from jax.experimental.pallas import tpu as pltpu
from jax.experimental.pallas import tpu_sc as plsc

N = 10000
E = 320000
F_IN = 128
H = 64
B = 64

NC = 2
NS = 16
NW = NC * NS
CH = 80
EPW = E // NW
NP = 10240
NPW = NP // NS
NIT = EPW // CH


def _deg_body(eidx_hbm, ones_hbm, zeros_hbm, out_hbm, didx_all, ones_v, acc, sem):
    c = lax.axis_index("c")
    s = lax.axis_index("s")
    w = c * NS + s
    pltpu.sync_copy(zeros_hbm.at[pl.ds(s * NPW, NPW)], acc.at[pl.ds(s * NPW, NPW)])
    pltpu.sync_copy(eidx_hbm.at[1].at[pl.ds(w * NIT, NIT)], didx_all)
    pltpu.sync_copy(ones_hbm, ones_v)
    plsc.subcore_barrier()

    def body(i, carry):
        pltpu.async_copy(ones_v, acc.at[didx_all.at[i]], sem, add=True)
        return carry

    lax.fori_loop(0, NIT, body, 0)

    def drain(i, carry):
        pltpu.make_async_copy(ones_v, acc.at[didx_all.at[i]], sem).wait()
        return carry

    lax.fori_loop(0, NIT, drain, 0)
    plsc.subcore_barrier()
    pltpu.sync_copy(acc.at[pl.ds(s * NPW, NPW)],
                    out_hbm.at[c].at[pl.ds(s * NPW, NPW)])


def _agg_body(hs_hbm, eidx_hbm, zeros_hbm, out_hbm,
              sidx_all, didx_all, rows0, rows1, acc, sem0, sem1):
    c = lax.axis_index("c")
    s = lax.axis_index("s")
    w = c * NS + s
    pltpu.sync_copy(zeros_hbm.at[pl.ds(s * NPW, NPW)], acc.at[pl.ds(s * NPW, NPW)])
    pltpu.sync_copy(eidx_hbm.at[0].at[pl.ds(w * NIT, NIT)], sidx_all)
    pltpu.sync_copy(eidx_hbm.at[1].at[pl.ds(w * NIT, NIT)], didx_all)
    plsc.subcore_barrier()

    def wait_gather(buf, sem):
        pltpu.make_async_copy(hs_hbm.at[sidx_all.at[0]], buf, sem).wait()

    pltpu.async_copy(hs_hbm.at[sidx_all.at[0]], rows0, sem0)

    def body(i, carry):
        even = lax.rem(i, 2) == 0

        @pl.when(even)
        def _():
            wait_gather(rows0, sem0)

            @pl.when(i + 1 < NIT)
            def _():
                pltpu.async_copy(hs_hbm.at[sidx_all.at[i + 1]], rows1, sem1)

            pltpu.sync_copy(rows0, acc.at[didx_all.at[i]], add=True)

        @pl.when(jnp.logical_not(even))
        def _():
            wait_gather(rows1, sem1)

            @pl.when(i + 1 < NIT)
            def _():
                pltpu.async_copy(hs_hbm.at[sidx_all.at[i + 1]], rows0, sem0)

            pltpu.sync_copy(rows1, acc.at[didx_all.at[i]], add=True)

        return carry

    lax.fori_loop(0, NIT, body, 0)
    plsc.subcore_barrier()
    pltpu.sync_copy(acc.at[pl.ds(s * NPW, NPW)],
                    out_hbm.at[c].at[pl.ds(s * NPW, NPW)])


@functools.cache
def _sc_calls():
    mesh = plsc.VectorSubcoreMesh(
        core_axis_name="c", subcore_axis_name="s",
        num_cores=NC, num_subcores=NS)
    params = pltpu.CompilerParams(use_tc_tiling_on_sc=False)
    deg_call = pl.kernel(
        _deg_body,
        out_type=jax.ShapeDtypeStruct((NC, NP, 16), jnp.float32),
        mesh=mesh,
        compiler_params=params,
        scratch_types=[
            pltpu.VMEM((NIT, CH), jnp.int32),
            pltpu.VMEM((CH, 16), jnp.float32),
            pltpu.VMEM_SHARED((NP, 16), jnp.float32),
            pltpu.SemaphoreType.DMA,
        ],
    )
    agg_call = pl.kernel(
        _agg_body,
        out_type=jax.ShapeDtypeStruct((NC, NP, H), jnp.float32),
        mesh=mesh,
        compiler_params=params,
        scratch_types=[
            pltpu.VMEM((NIT, CH), jnp.int32),
            pltpu.VMEM((NIT, CH), jnp.int32),
            pltpu.VMEM((CH, H), jnp.float32),
            pltpu.VMEM((CH, H), jnp.float32),
            pltpu.VMEM_SHARED((NP, H), jnp.float32),
            pltpu.SemaphoreType.DMA,
            pltpu.SemaphoreType.DMA,
        ],
    )
    return deg_call, agg_call


def _dot(a, b):
    return lax.dot_general(a, b, (((1,), (0,)), ((), ())),
                           preferred_element_type=jnp.float32)


def _prep_body(x_ref, dega_ref, degb_ref, w_ref, dinv_ref, hs_ref):
    deg = dega_ref[0:N, 0:1] + degb_ref[0:N, 0:1] + 1.0
    dinv = lax.rsqrt(deg)
    dinv_ref[...] = dinv
    hs_ref[0:N, :] = dinv * _dot(x_ref[...], w_ref[...])


def _update_body(acc_ref, hs_ref, dinv_ref, b_ref, w_ref, out_ref):
    dinv = dinv_ref[...]
    t = jnp.tanh(dinv * (acc_ref[0, 0:N, :] + acc_ref[1, 0:N, :]
                         + hs_ref[0:N, :]) + b_ref[...])
    out_ref[0:N, :] = dinv * _dot(t, w_ref[...])


def _final_body(acc_ref, hs_ref, dinv_ref, b_ref, batch_ref,
                wout_ref, bout_ref, out_ref, t_ref, gmax_ref, gsum_ref, cnt_ref):
    dinv = dinv_ref[...]
    t_ref[...] = jnp.tanh(
        dinv * (acc_ref[0, 0:N, :] + acc_ref[1, 0:N, :]
                + hs_ref[0:N, :]) + b_ref[...])

    def body(b, carry):
        t = t_ref[...]
        mask = batch_ref[...] == b
        gmax_ref[pl.ds(b, 1), :] = jnp.max(
            jnp.where(mask, t, -jnp.inf), axis=0, keepdims=True)
        gsum_ref[pl.ds(b, 1), :] = jnp.sum(
            jnp.where(mask, t, 0.0), axis=0, keepdims=True)
        cnt_ref[pl.ds(b, 1), :] = jnp.sum(
            mask.astype(jnp.float32), axis=0, keepdims=True)
        return carry

    lax.fori_loop(0, B, body, 0)
    gmean = gsum_ref[...] / jnp.maximum(cnt_ref[...], 1.0)
    pooled = jnp.concatenate([gmax_ref[...], gmean], axis=1)
    out_ref[...] = _dot(pooled, wout_ref[...]) + bout_ref[...]


_prep_call = pl.pallas_call(
    _prep_body,
    out_shape=[jax.ShapeDtypeStruct((N, 1), jnp.float32),
               jax.ShapeDtypeStruct((NP, H), jnp.float32)],
)

_update_call = pl.pallas_call(
    _update_body,
    out_shape=jax.ShapeDtypeStruct((NP, H), jnp.float32),
)

_final_call = pl.pallas_call(
    _final_body,
    out_shape=jax.ShapeDtypeStruct((B, 1), jnp.float32),
    scratch_shapes=[
        pltpu.VMEM((N, H), jnp.float32),
        pltpu.VMEM((B, H), jnp.float32),
        pltpu.VMEM((B, H), jnp.float32),
        pltpu.VMEM((B, 1), jnp.float32),
    ],
)


def kernel(x, edge_index, batch_index, W0, b0, W1, b1, W2, b2, W_out, b_out):
    eidx = edge_index.reshape(2, E // CH, CH)
    zeros_h = jnp.zeros((NP, H), jnp.float32)
    zeros_16 = jnp.zeros((NP, 16), jnp.float32)
    ones_16 = jnp.ones((CH, 16), jnp.float32)
    batch2d = batch_index.reshape(N, 1)

    deg_call, agg_call = _sc_calls()

    deg = deg_call(eidx, ones_16, zeros_16)
    dinv, hs = _prep_call(x, deg[0], deg[1], W0)

    acc = agg_call(hs, eidx, zeros_h)
    hs = _update_call(acc, hs, dinv, b0.reshape(1, H), W1)

    acc = agg_call(hs, eidx, zeros_h)
    hs = _update_call(acc, hs, dinv, b1.reshape(1, H), W2)

    acc = agg_call(hs, eidx, zeros_h)
    out = _final_call(acc, hs, dinv, b2.reshape(1, H), batch2d,
                      W_out, b_out.reshape(1, 1))
    return out


# speedup vs baseline: 1.3009x; 1.3009x over previous
import functools

import jax
import jax.numpy as jnp
from jax import lax
from jax.experimental import pallas as pl
from jax.experimental.pallas import tpu as pltpu
from jax.experimental.pallas import tpu_sc as plsc

N = 10000
E = 320000
F_IN = 128
H = 64
B = 64

NC = 2
NS = 16
NW = NC * NS
CH = 80
EPW = E // NW
NP = 10240
NPW = NP // NS
NIT = EPW // CH


def _deg_body(eidx_hbm, ones_hbm, zeros_hbm, out_hbm, didx_all, ones_v, acc, sem):
    c = lax.axis_index("c")
    s = lax.axis_index("s")
    w = c * NS + s
    pltpu.sync_copy(zeros_hbm.at[pl.ds(s * NPW, NPW)], acc.at[pl.ds(s * NPW, NPW)])
    pltpu.sync_copy(eidx_hbm.at[1].at[pl.ds(w * NIT, NIT)], didx_all)
    pltpu.sync_copy(ones_hbm, ones_v)
    plsc.subcore_barrier()

    def body(i, carry):
        pltpu.async_copy(ones_v, acc.at[didx_all.at[i]], sem, add=True)
        return carry

    lax.fori_loop(0, NIT, body, 0)

    def drain(i, carry):
        pltpu.make_async_copy(ones_v, acc.at[didx_all.at[i]], sem).wait()
        return carry

    lax.fori_loop(0, NIT, drain, 0)
    plsc.subcore_barrier()
    pltpu.sync_copy(acc.at[pl.ds(s * NPW, NPW)],
                    out_hbm.at[c].at[pl.ds(s * NPW, NPW)])


def _agg_body(hs_hbm, eidx_hbm, zeros_hbm, out_hbm,
              sidx_all, didx_all, rows0, rows1, hs_s, acc, sem0, sem1):
    c = lax.axis_index("c")
    s = lax.axis_index("s")
    w = c * NS + s
    pltpu.sync_copy(zeros_hbm.at[pl.ds(s * NPW, NPW)], acc.at[pl.ds(s * NPW, NPW)])
    pltpu.sync_copy(hs_hbm.at[pl.ds(s * NPW, NPW)], hs_s.at[pl.ds(s * NPW, NPW)])
    pltpu.sync_copy(eidx_hbm.at[0].at[pl.ds(w * NIT, NIT)], sidx_all)
    pltpu.sync_copy(eidx_hbm.at[1].at[pl.ds(w * NIT, NIT)], didx_all)
    plsc.subcore_barrier()

    def wait_gather(buf, sem):
        pltpu.make_async_copy(hs_s.at[sidx_all.at[0]], buf, sem).wait()

    pltpu.async_copy(hs_s.at[sidx_all.at[0]], rows0, sem0)

    def body(i, carry):
        even = lax.rem(i, 2) == 0

        @pl.when(even)
        def _():
            wait_gather(rows0, sem0)

            @pl.when(i + 1 < NIT)
            def _():
                pltpu.async_copy(hs_s.at[sidx_all.at[i + 1]], rows1, sem1)

            pltpu.sync_copy(rows0, acc.at[didx_all.at[i]], add=True)

        @pl.when(jnp.logical_not(even))
        def _():
            wait_gather(rows1, sem1)

            @pl.when(i + 1 < NIT)
            def _():
                pltpu.async_copy(hs_s.at[sidx_all.at[i + 1]], rows0, sem0)

            pltpu.sync_copy(rows1, acc.at[didx_all.at[i]], add=True)

        return carry

    lax.fori_loop(0, NIT, body, 0)
    plsc.subcore_barrier()
    pltpu.sync_copy(acc.at[pl.ds(s * NPW, NPW)],
                    out_hbm.at[c].at[pl.ds(s * NPW, NPW)])


@functools.cache
def _sc_calls():
    mesh = plsc.VectorSubcoreMesh(
        core_axis_name="c", subcore_axis_name="s",
        num_cores=NC, num_subcores=NS)
    params = pltpu.CompilerParams(use_tc_tiling_on_sc=False)
    deg_call = pl.kernel(
        _deg_body,
        out_type=jax.ShapeDtypeStruct((NC, NP, 16), jnp.float32),
        mesh=mesh,
        compiler_params=params,
        scratch_types=[
            pltpu.VMEM((NIT, CH), jnp.int32),
            pltpu.VMEM((CH, 16), jnp.float32),
            pltpu.VMEM_SHARED((NP, 16), jnp.float32),
            pltpu.SemaphoreType.DMA,
        ],
    )
    agg_call = pl.kernel(
        _agg_body,
        out_type=jax.ShapeDtypeStruct((NC, NP, H), jnp.float32),
        mesh=mesh,
        compiler_params=params,
        scratch_types=[
            pltpu.VMEM((NIT, CH), jnp.int32),
            pltpu.VMEM((NIT, CH), jnp.int32),
            pltpu.VMEM((CH, H), jnp.float32),
            pltpu.VMEM((CH, H), jnp.float32),
            pltpu.VMEM_SHARED((NP, H), jnp.float32),
            pltpu.VMEM_SHARED((NP, H), jnp.float32),
            pltpu.SemaphoreType.DMA,
            pltpu.SemaphoreType.DMA,
        ],
    )
    return deg_call, agg_call


def _dot(a, b):
    return lax.dot_general(a, b, (((1,), (0,)), ((), ())),
                           preferred_element_type=jnp.float32)


def _prep_body(x_ref, dega_ref, degb_ref, w_ref, dinv_ref, hs_ref):
    deg = dega_ref[0:N, 0:1] + degb_ref[0:N, 0:1] + 1.0
    dinv = lax.rsqrt(deg)
    dinv_ref[...] = dinv
    hs_ref[0:N, :] = dinv * _dot(x_ref[...], w_ref[...])


def _update_body(acc_ref, hs_ref, dinv_ref, b_ref, w_ref, out_ref):
    dinv = dinv_ref[...]
    t = jnp.tanh(dinv * (acc_ref[0, 0:N, :] + acc_ref[1, 0:N, :]
                         + hs_ref[0:N, :]) + b_ref[...])
    out_ref[0:N, :] = dinv * _dot(t, w_ref[...])


def _final_body(acc_ref, hs_ref, dinv_ref, b_ref, batch_ref, batchr_ref,
                wout_ref, bout_ref, out_ref, t_ref, gmax_ref):
    dinv = dinv_ref[...]
    t_ref[...] = jnp.tanh(
        dinv * (acc_ref[0, 0:N, :] + acc_ref[1, 0:N, :]
                + hs_ref[0:N, :]) + b_ref[...])

    oh = (batchr_ref[...] ==
          lax.broadcasted_iota(jnp.int32, (B, N), 0)).astype(jnp.float32)
    gsum = _dot(oh, t_ref[...])
    cnt = _dot(oh, jnp.ones((N, 1), jnp.float32))

    def body(b, carry):
        pen = (batch_ref[...] == b).astype(jnp.float32) * 1e30 - 1e30
        gmax_ref[pl.ds(b, 1), :] = jnp.max(
            t_ref[...] + pen, axis=0, keepdims=True)
        return carry

    lax.fori_loop(0, B, body, 0)
    gmean = gsum / jnp.maximum(cnt, 1.0)
    pooled = jnp.concatenate([gmax_ref[...], gmean], axis=1)
    out_ref[...] = _dot(pooled, wout_ref[...]) + bout_ref[...]


_prep_call = pl.pallas_call(
    _prep_body,
    out_shape=[jax.ShapeDtypeStruct((N, 1), jnp.float32),
               jax.ShapeDtypeStruct((NP, H), jnp.float32)],
)

_update_call = pl.pallas_call(
    _update_body,
    out_shape=jax.ShapeDtypeStruct((NP, H), jnp.float32),
)

_final_call = pl.pallas_call(
    _final_body,
    out_shape=jax.ShapeDtypeStruct((B, 1), jnp.float32),
    scratch_shapes=[
        pltpu.VMEM((N, H), jnp.float32),
        pltpu.VMEM((B, H), jnp.float32),
    ],
)


def kernel(x, edge_index, batch_index, W0, b0, W1, b1, W2, b2, W_out, b_out):
    eidx = edge_index.reshape(2, E // CH, CH)
    zeros_h = jnp.zeros((NP, H), jnp.float32)
    zeros_16 = jnp.zeros((NP, 16), jnp.float32)
    ones_16 = jnp.ones((CH, 16), jnp.float32)
    batch2d = batch_index.reshape(N, 1)

    deg_call, agg_call = _sc_calls()

    deg = deg_call(eidx, ones_16, zeros_16)
    dinv, hs = _prep_call(x, deg[0], deg[1], W0)

    acc = agg_call(hs, eidx, zeros_h)
    hs = _update_call(acc, hs, dinv, b0.reshape(1, H), W1)

    acc = agg_call(hs, eidx, zeros_h)
    hs = _update_call(acc, hs, dinv, b1.reshape(1, H), W2)

    acc = agg_call(hs, eidx, zeros_h)
    out = _final_call(acc, hs, dinv, b2.reshape(1, H), batch2d,
                      batch_index.reshape(1, N), W_out, b_out.reshape(1, 1))
    return out


# speedup vs baseline: 1.5499x; 1.1914x over previous
import functools

import jax
import jax.numpy as jnp
from jax import lax
from jax.experimental import pallas as pl
from jax.experimental.pallas import tpu as pltpu
from jax.experimental.pallas import tpu_sc as plsc

N = 10000
E = 320000
F_IN = 128
H = 64
B = 64

NC = 2
NS = 16
NW = NC * NS
CH = 80
EPW = E // NW
NP = 10240
NPW = NP // NS
NIT = EPW // CH


def _deg_body(eidx_hbm, ones_hbm, zeros_hbm, out_hbm, didx_all, ones_v, acc, sem):
    c = lax.axis_index("c")
    s = lax.axis_index("s")
    w = c * NS + s
    pltpu.sync_copy(zeros_hbm.at[pl.ds(s * NPW, NPW)], acc.at[pl.ds(s * NPW, NPW)])
    pltpu.sync_copy(eidx_hbm.at[1].at[pl.ds(w * NIT, NIT)], didx_all)
    pltpu.sync_copy(ones_hbm, ones_v)
    plsc.subcore_barrier()

    def body(i, carry):
        pltpu.async_copy(ones_v, acc.at[didx_all.at[i]], sem, add=True)
        return carry

    lax.fori_loop(0, NIT, body, 0)

    def drain(i, carry):
        pltpu.make_async_copy(ones_v, acc.at[didx_all.at[i]], sem).wait()
        return carry

    lax.fori_loop(0, NIT, drain, 0)
    plsc.subcore_barrier()
    pltpu.sync_copy(acc.at[pl.ds(s * NPW, NPW)],
                    out_hbm.at[c].at[pl.ds(s * NPW, NPW)])


def _agg_body(hs_hbm, eidx_hbm, zeros_hbm, out_hbm,
              sidx_all, didx_all, rows0, rows1, hs_s, acc, sem0, sem1):
    c = lax.axis_index("c")
    s = lax.axis_index("s")
    w = c * NS + s
    pltpu.sync_copy(zeros_hbm.at[pl.ds(s * NPW, NPW)], acc.at[pl.ds(s * NPW, NPW)])
    pltpu.sync_copy(hs_hbm.at[pl.ds(s * NPW, NPW)], hs_s.at[pl.ds(s * NPW, NPW)])
    pltpu.sync_copy(eidx_hbm.at[0].at[pl.ds(w * NIT, NIT)], sidx_all)
    pltpu.sync_copy(eidx_hbm.at[1].at[pl.ds(w * NIT, NIT)], didx_all)
    plsc.subcore_barrier()

    def wait_gather(buf, sem):
        pltpu.make_async_copy(hs_s.at[sidx_all.at[0]], buf, sem).wait()

    pltpu.async_copy(hs_s.at[sidx_all.at[0]], rows0, sem0)

    def body(i, carry):
        even = lax.rem(i, 2) == 0

        @pl.when(even)
        def _():
            wait_gather(rows0, sem0)

            @pl.when(i + 1 < NIT)
            def _():
                pltpu.async_copy(hs_s.at[sidx_all.at[i + 1]], rows1, sem1)

            pltpu.sync_copy(rows0, acc.at[didx_all.at[i]], add=True)

        @pl.when(jnp.logical_not(even))
        def _():
            wait_gather(rows1, sem1)

            @pl.when(i + 1 < NIT)
            def _():
                pltpu.async_copy(hs_s.at[sidx_all.at[i + 1]], rows0, sem0)

            pltpu.sync_copy(rows1, acc.at[didx_all.at[i]], add=True)

        return carry

    lax.fori_loop(0, NIT, body, 0)
    plsc.subcore_barrier()
    pltpu.sync_copy(acc.at[pl.ds(s * NPW, NPW)],
                    out_hbm.at[c].at[pl.ds(s * NPW, NPW)])


@functools.cache
def _sc_calls():
    mesh = plsc.VectorSubcoreMesh(
        core_axis_name="c", subcore_axis_name="s",
        num_cores=NC, num_subcores=NS)
    params = pltpu.CompilerParams(use_tc_tiling_on_sc=False)
    deg_call = pl.kernel(
        _deg_body,
        out_type=jax.ShapeDtypeStruct((NC, NP, 16), jnp.float32),
        mesh=mesh,
        compiler_params=params,
        scratch_types=[
            pltpu.VMEM((NIT, CH), jnp.int32),
            pltpu.VMEM((CH, 16), jnp.float32),
            pltpu.VMEM_SHARED((NP, 16), jnp.float32),
            pltpu.SemaphoreType.DMA,
        ],
    )
    agg_call = pl.kernel(
        _agg_body,
        out_type=jax.ShapeDtypeStruct((NC, NP, H), jnp.float32),
        mesh=mesh,
        compiler_params=params,
        scratch_types=[
            pltpu.VMEM((NIT, CH), jnp.int32),
            pltpu.VMEM((NIT, CH), jnp.int32),
            pltpu.VMEM((CH, H), jnp.float32),
            pltpu.VMEM((CH, H), jnp.float32),
            pltpu.VMEM_SHARED((NP, H), jnp.float32),
            pltpu.VMEM_SHARED((NP, H), jnp.float32),
            pltpu.SemaphoreType.DMA,
            pltpu.SemaphoreType.DMA,
        ],
    )
    return deg_call, agg_call


def _dot(a, b):
    return lax.dot_general(a, b, (((1,), (0,)), ((), ())),
                           preferred_element_type=jnp.float32)


def _prep_body(x_ref, dega_ref, degb_ref, w_ref, dinv_ref, hs_ref):
    deg = dega_ref[0:N, 0:1] + degb_ref[0:N, 0:1] + 1.0
    dinv = lax.rsqrt(deg)
    dinv_ref[...] = dinv
    hs_ref[0:N, :] = dinv * _dot(x_ref[...], w_ref[...])


def _update_body(acc_ref, hs_ref, dinv_ref, b_ref, w_ref, out_ref):
    dinv = dinv_ref[...]
    t = jnp.tanh(dinv * (acc_ref[0, 0:N, :] + acc_ref[1, 0:N, :]
                         + hs_ref[0:N, :]) + b_ref[...])
    out_ref[0:N, :] = dinv * _dot(t, w_ref[...])


WIN = 1024
NT = 11264


def _final_body(acc_ref, hs_ref, dinv_ref, b_ref, batch_ref, batchr_ref,
                starts_ref, wout_ref, bout_ref, out_ref, t_ref, gmax_ref):
    dinv = dinv_ref[...]
    t_ref[0:N, :] = jnp.tanh(
        dinv * (acc_ref[0, 0:N, :] + acc_ref[1, 0:N, :]
                + hs_ref[0:N, :]) + b_ref[...])

    oh = (batchr_ref[...] ==
          lax.broadcasted_iota(jnp.int32, (B, N), 0)).astype(jnp.float32)
    gsum = _dot(oh, t_ref[0:N, :])
    cnt = _dot(oh, jnp.ones((N, 1), jnp.float32))

    def seg(b, carry):
        s0 = starts_ref[b]
        e0 = starts_ref[b + 1]
        w0 = 8 * lax.div(s0, 8)
        nch = lax.div(e0 - w0 + (WIN - 1), WIN)

        def chunk(j, acc):
            off = pl.multiple_of(w0 + j * WIN, 8)
            tw = t_ref[pl.ds(off, WIN), :]
            mask = batch_ref[pl.ds(off, WIN), :] == b
            return jnp.maximum(
                acc, jnp.max(jnp.where(mask, tw, -1e30), axis=0, keepdims=True))

        acc = lax.fori_loop(0, nch, chunk,
                            jnp.full((1, H), -1e30, jnp.float32))
        gmax_ref[pl.ds(b, 1), :] = acc
        return carry

    lax.fori_loop(0, B, seg, 0)
    gmean = gsum / jnp.maximum(cnt, 1.0)
    pooled = jnp.concatenate([gmax_ref[...], gmean], axis=1)
    out_ref[...] = _dot(pooled, wout_ref[...]) + bout_ref[...]


_prep_call = pl.pallas_call(
    _prep_body,
    out_shape=[jax.ShapeDtypeStruct((N, 1), jnp.float32),
               jax.ShapeDtypeStruct((NP, H), jnp.float32)],
)

_update_call = pl.pallas_call(
    _update_body,
    out_shape=jax.ShapeDtypeStruct((NP, H), jnp.float32),
)

_final_call = pl.pallas_call(
    _final_body,
    out_shape=jax.ShapeDtypeStruct((B, 1), jnp.float32),
    in_specs=[pl.BlockSpec(), pl.BlockSpec(), pl.BlockSpec(), pl.BlockSpec(),
              pl.BlockSpec(), pl.BlockSpec(),
              pl.BlockSpec(memory_space=pltpu.SMEM),
              pl.BlockSpec(), pl.BlockSpec()],
    scratch_shapes=[
        pltpu.VMEM((NT, H), jnp.float32),
        pltpu.VMEM((B, H), jnp.float32),
    ],
)


def kernel(x, edge_index, batch_index, W0, b0, W1, b1, W2, b2, W_out, b_out):
    eidx = edge_index.reshape(2, E // CH, CH)
    zeros_h = jnp.zeros((NP, H), jnp.float32)
    zeros_16 = jnp.zeros((NP, 16), jnp.float32)
    ones_16 = jnp.ones((CH, 16), jnp.float32)

    deg_call, agg_call = _sc_calls()

    deg = deg_call(eidx, ones_16, zeros_16)
    dinv, hs = _prep_call(x, deg[0], deg[1], W0)

    acc = agg_call(hs, eidx, zeros_h)
    hs = _update_call(acc, hs, dinv, b0.reshape(1, H), W1)

    acc = agg_call(hs, eidx, zeros_h)
    hs = _update_call(acc, hs, dinv, b1.reshape(1, H), W2)

    acc = agg_call(hs, eidx, zeros_h)
    starts = jnp.searchsorted(
        batch_index, jnp.arange(B + 1, dtype=jnp.int32)).astype(jnp.int32)
    batch_nt = jnp.concatenate(
        [batch_index, jnp.full((NT - N,), -1, jnp.int32)]).reshape(NT, 1)
    out = _final_call(acc, hs, dinv, b2.reshape(1, H), batch_nt,
                      batch_index.reshape(1, N), starts,
                      W_out, b_out.reshape(1, 1))
    return out
